# Initial kernel scaffold; baseline (speedup 1.0000x reference)
import jax
import jax.numpy as jnp
from jax.experimental import pallas as pl


def kernel(edge_feats, node_feats, edge_index, We, Ws, Wd, b1, W2, b2, gamma, beta):
    raise NotImplementedError("write your pallas kernel here")


# speedup vs baseline: 3.2701x; 3.2701x over previous
import functools

import jax
import jax.numpy as jnp
from jax import lax
from jax.experimental import pallas as pl
from jax.experimental.pallas import tpu as pltpu

try:
    from jax.experimental.pallas import tpu_sc as plsc
    _HAS_SC = True
except ImportError:
    _HAS_SC = False

N = 10000
E = 320000
D = 128
H = 256


_NW = 32
_EPW = E // _NW
_CH = 80
_NCH = _EPW // _CH


def _sc_gather_build():
    mesh = plsc.VectorSubcoreMesh(core_axis_name="c", subcore_axis_name="s")

    @functools.partial(
        pl.kernel,
        mesh=mesh,
        out_type=[
            jax.ShapeDtypeStruct((E, D), jnp.float32),
            jax.ShapeDtypeStruct((E, D), jnp.float32),
        ],
        scratch_types=[
            pltpu.VMEM((_EPW,), jnp.int32),
            pltpu.VMEM((_EPW,), jnp.int32),
            pltpu.VMEM((_CH, D), jnp.float32),
            pltpu.VMEM((_CH, D), jnp.float32),
            pltpu.SemaphoreType.DMA,
            pltpu.SemaphoreType.DMA,
        ],
    )
    def sc_gather(nf_hbm, src_hbm, dst_hbm, out_s_hbm, out_d_hbm,
                  idx_s, idx_d, rows_s, rows_d, sem_s, sem_d):
        wid = lax.axis_index("s") * 2 + lax.axis_index("c")
        base = wid * _EPW
        pltpu.sync_copy(src_hbm.at[pl.ds(base, _EPW)], idx_s)
        pltpu.sync_copy(dst_hbm.at[pl.ds(base, _EPW)], idx_d)

        def body(j, carry):
            off = j * _CH
            cs = pltpu.async_copy(nf_hbm.at[idx_s.at[pl.ds(off, _CH)]],
                                  rows_s, sem_s)
            cd = pltpu.async_copy(nf_hbm.at[idx_d.at[pl.ds(off, _CH)]],
                                  rows_d, sem_d)
            cs.wait()
            pltpu.sync_copy(rows_s, out_s_hbm.at[pl.ds(base + off, _CH)])
            cd.wait()
            pltpu.sync_copy(rows_d, out_d_hbm.at[pl.ds(base + off, _CH)])
            return carry

        lax.fori_loop(0, _NCH, body, 0)

    return sc_gather


_BE = 1280


def _mlp_body(ef_ref, gs_ref, gd_ref, wet_ref, wst_ref, wdt_ref, b1_ref,
              w2t_ref, b2_ref, gamma_ref, beta_ref, out_ref):
    ef = ef_ref[...]
    h = jnp.dot(ef, wet_ref[...], preferred_element_type=jnp.float32)
    h += jnp.dot(gs_ref[...], wst_ref[...], preferred_element_type=jnp.float32)
    h += jnp.dot(gd_ref[...], wdt_ref[...], preferred_element_type=jnp.float32)
    h += b1_ref[...]
    h = h * jax.nn.sigmoid(h)
    y = jnp.dot(h, w2t_ref[...], preferred_element_type=jnp.float32)
    y += b2_ref[...]
    mu = jnp.mean(y, axis=-1, keepdims=True)
    var = jnp.mean((y - mu) ** 2, axis=-1, keepdims=True)
    y = (y - mu) * lax.rsqrt(var + 1e-5) * gamma_ref[...] + beta_ref[...]
    out_ref[...] = y + ef


def _mlp_call(ef, gs, gd, WeT, WsT, WdT, b1, W2T, b2, gamma, beta,
              interpret=False):
    grid = (E // _BE,)
    eb = pl.BlockSpec((_BE, D), lambda i: (i, 0))
    full = lambda shape: pl.BlockSpec(shape, lambda i: tuple(0 for _ in shape))
    return pl.pallas_call(
        _mlp_body,
        grid=grid,
        in_specs=[
            eb, eb, eb,
            full((D, H)), full((D, H)), full((D, H)), full((1, H)),
            full((H, D)), full((1, D)), full((1, D)), full((1, D)),
        ],
        out_specs=eb,
        out_shape=jax.ShapeDtypeStruct((E, D), jnp.float32),
        interpret=interpret,
    )(ef, gs, gd, WeT, WsT, WdT, b1, W2T, b2, gamma, beta)


def kernel(edge_feats, node_feats, edge_index, We, Ws, Wd, b1, W2, b2,
           gamma, beta):
    src = edge_index[0].astype(jnp.int32)
    dst = edge_index[1].astype(jnp.int32)
    gs, gd = _sc_gather_build()(node_feats, src, dst)
    out = _mlp_call(
        edge_feats, gs, gd,
        We.T, Ws.T, Wd.T, b1.reshape(1, H),
        W2.T, b2.reshape(1, D), gamma.reshape(1, D), beta.reshape(1, D),
    )
    return (out, node_feats)


# speedup vs baseline: 3.5979x; 1.1002x over previous
import functools

import jax
import jax.numpy as jnp
from jax import lax
from jax.experimental import pallas as pl
from jax.experimental.pallas import tpu as pltpu
from jax.experimental.pallas import tpu_sc as plsc

N = 10000
E = 320000
D = 128
H = 256
DW = D // 2


_NW = 32
_EPW = E // _NW
_CH = 80
_NCH = _EPW // _CH


def _sc_gather_build():
    mesh = plsc.VectorSubcoreMesh(core_axis_name="c", subcore_axis_name="s")

    @functools.partial(
        pl.kernel,
        mesh=mesh,
        out_type=[
            jax.ShapeDtypeStruct((E, D), jnp.float32),
            jax.ShapeDtypeStruct((E, D), jnp.float32),
        ],
        scratch_types=[
            pltpu.VMEM((_EPW,), jnp.int32),
            pltpu.VMEM((_EPW,), jnp.int32),
            pltpu.VMEM((2, _CH, D), jnp.float32),
            pltpu.VMEM((2, _CH, D), jnp.float32),
            pltpu.SemaphoreType.DMA,
            pltpu.SemaphoreType.DMA,
        ],
    )
    def sc_gather(nf_hbm, src_hbm, dst_hbm, out_s_hbm, out_d_hbm,
                  idx_s, idx_d, rows_s, rows_d, sem_s, sem_d):
        wid = lax.axis_index("s") * 2 + lax.axis_index("c")
        base = wid * _EPW
        pltpu.sync_copy(src_hbm.at[pl.ds(base, _EPW)], idx_s)
        pltpu.sync_copy(dst_hbm.at[pl.ds(base, _EPW)], idx_d)

        def fire(j, slot):
            off = j * _CH
            cs = pltpu.async_copy(nf_hbm.at[idx_s.at[pl.ds(off, _CH)]],
                                  rows_s.at[slot], sem_s)
            cd = pltpu.async_copy(nf_hbm.at[idx_d.at[pl.ds(off, _CH)]],
                                  rows_d.at[slot], sem_d)
            return cs, cd

        def drain(j, slot):
            off = j * _CH
            pltpu.make_async_copy(nf_hbm.at[idx_s.at[pl.ds(off, _CH)]],
                                  rows_s.at[slot], sem_s).wait()
            pltpu.sync_copy(rows_s.at[slot], out_s_hbm.at[pl.ds(base + off, _CH)])
            pltpu.make_async_copy(nf_hbm.at[idx_d.at[pl.ds(off, _CH)]],
                                  rows_d.at[slot], sem_d).wait()
            pltpu.sync_copy(rows_d.at[slot], out_d_hbm.at[pl.ds(base + off, _CH)])

        fire(0, 0)

        def body(j, carry):
            slot = lax.rem(j, 2)

            @pl.when(j + 1 < _NCH)
            def _():
                fire(j + 1, 1 - slot)

            drain(j, slot)
            return carry

        lax.fori_loop(0, _NCH, body, 0)

    return sc_gather


_BE = 1280


def _mlp_body(ef_ref, gs_ref, gd_ref, w1t_ref, b1_ref,
              w2t_ref, b2_ref, gamma_ref, beta_ref, out_ref):
    ef = ef_ref[...]
    x = jnp.concatenate(
        [ef, gs_ref[...], gd_ref[...]], axis=-1).astype(jnp.bfloat16)
    h = jnp.dot(x, w1t_ref[...], preferred_element_type=jnp.float32)
    h += b1_ref[...]
    h = h * jax.nn.sigmoid(h)
    y = jnp.dot(h.astype(jnp.bfloat16), w2t_ref[...],
                preferred_element_type=jnp.float32)
    y += b2_ref[...]
    mu = jnp.mean(y, axis=-1, keepdims=True)
    var = jnp.mean((y - mu) ** 2, axis=-1, keepdims=True)
    y = (y - mu) * lax.rsqrt(var + 1e-5) * gamma_ref[...] + beta_ref[...]
    out_ref[...] = y + ef


def _mlp_call(ef, gs, gd, W1T, b1, W2T, b2, gamma, beta, interpret=False):
    grid = (E // _BE,)
    eb = pl.BlockSpec((_BE, D), lambda i: (i, 0))
    gb = pl.BlockSpec((_BE, D), lambda i: (i, 0))
    full = lambda shape: pl.BlockSpec(shape, lambda i: tuple(0 for _ in shape))
    return pl.pallas_call(
        _mlp_body,
        grid=grid,
        in_specs=[
            eb, gb, gb,
            full((3 * D, H)), full((1, H)),
            full((H, D)), full((1, D)), full((1, D)), full((1, D)),
        ],
        out_specs=eb,
        out_shape=jax.ShapeDtypeStruct((E, D), jnp.float32),
        interpret=interpret,
    )(ef, gs, gd, W1T, b1, W2T, b2, gamma, beta)


def kernel(edge_feats, node_feats, edge_index, We, Ws, Wd, b1, W2, b2,
           gamma, beta):
    src = edge_index[0].astype(jnp.int32)
    dst = edge_index[1].astype(jnp.int32)
    gs, gd = _sc_gather_build()(node_feats, src, dst)
    W1T = jnp.concatenate(
        [We.T, Ws.T, Wd.T], axis=0).astype(jnp.bfloat16)
    out = _mlp_call(
        edge_feats, gs, gd,
        W1T, b1.reshape(1, H),
        W2.T.astype(jnp.bfloat16), b2.reshape(1, D),
        gamma.reshape(1, D), beta.reshape(1, D),
    )
    return (out, node_feats)
